# initial kernel scaffold (unmeasured)
import jax
import jax.numpy as jnp
from jax import lax
from jax.experimental import pallas as pl
from jax.experimental.pallas import tpu as pltpu

N_DEV = 4
B = 2
SQ = 512
SKV = 512
HG = 8
DH = 64
D_MODEL = 768
BLK = 64


def kernel(x, Wq, K_ext, V_ext, Wo):
    K_r = jnp.transpose(K_ext.reshape(B, SKV, N_DEV, HG, DH), (2, 0, 3, 1, 4))
    V_r = jnp.transpose(V_ext.reshape(B, SKV, N_DEV, HG, DH), (2, 0, 3, 1, 4))

    def body(x_ref, wq_ref, k_ref, v_ref, wo_ref, out_ref,
             comm_wq, comm_wo, send_wq, recv_wq, send_wo, recv_wo):
        my = lax.axis_index("i")
        left = (my + N_DEV - 1) % N_DEV
        right = (my + 1) % N_DEV

        barrier_sem = pltpu.get_barrier_semaphore()
        for nbr in (left, right):
            pl.semaphore_signal(
                barrier_sem, inc=1,
                device_id=(nbr,), device_id_type=pl.DeviceIdType.MESH,
            )
        pl.semaphore_wait(barrier_sem, 2)

        comm_wq[0] = wq_ref[...]
        comm_wo[0] = wo_ref[...]

        ri = lax.broadcasted_iota(jnp.int32, (SQ, SKV), 0)
        ci = lax.broadcasted_iota(jnp.int32, (SQ, SKV), 1)
        qb = my * (SQ // BLK) + ri // BLK
        kb = ci // BLK
        mask = (qb == kb) | (kb == 0) | ((qb + kb) % 3 == 0)
        bias = jnp.where(mask, 0.0, -1e9).astype(jnp.float32)

        def contribution(slot, origin):
            wq = comm_wq[slot]
            wo = comm_wo[slot]
            for b in range(B):
                q = jnp.dot(x_ref[b], wq, preferred_element_type=jnp.float32)
                acc = None
                for hh in range(HG):
                    qh = q[:, hh * DH:(hh + 1) * DH]
                    kh = k_ref[origin, b, hh]
                    vh = v_ref[origin, b, hh]
                    s = lax.dot_general(
                        qh, kh, (((1,), (1,)), ((), ())),
                        preferred_element_type=jnp.float32,
                    ) * 0.125 + bias
                    m = jnp.max(s, axis=1, keepdims=True)
                    e = jnp.exp(s - m)
                    w = e / jnp.sum(e, axis=1, keepdims=True)
                    ctx = jnp.dot(w, vh, preferred_element_type=jnp.float32)
                    part = jnp.dot(ctx, wo[hh * DH:(hh + 1) * DH, :],
                                   preferred_element_type=jnp.float32)
                    acc = part if acc is None else acc + part
                if slot == 0:
                    out_ref[b] = acc
                else:
                    out_ref[b] = out_ref[b] + acc

        for h in range(N_DEV - 1):
            rq = pltpu.make_async_remote_copy(
                src_ref=comm_wq.at[h], dst_ref=comm_wq.at[h + 1],
                send_sem=send_wq.at[h], recv_sem=recv_wq.at[h],
                device_id=(right,), device_id_type=pl.DeviceIdType.MESH,
            )
            ro = pltpu.make_async_remote_copy(
                src_ref=comm_wo.at[h], dst_ref=comm_wo.at[h + 1],
                send_sem=send_wo.at[h], recv_sem=recv_wo.at[h],
                device_id=(right,), device_id_type=pl.DeviceIdType.MESH,
            )
            rq.start()
            ro.start()
            contribution(h, (my + N_DEV - h) % N_DEV)
            rq.wait()
            ro.wait()

        contribution(N_DEV - 1, (my + 1) % N_DEV)

    return pl.pallas_call(
        body,
        out_shape=jax.ShapeDtypeStruct((B, SQ, D_MODEL), jnp.float32),
        in_specs=[pl.BlockSpec(memory_space=pltpu.VMEM)] * 5,
        out_specs=pl.BlockSpec(memory_space=pltpu.VMEM),
        scratch_shapes=[
            pltpu.VMEM((N_DEV, D_MODEL, HG * DH), jnp.float32),
            pltpu.VMEM((N_DEV, HG * DH, D_MODEL), jnp.float32),
            pltpu.SemaphoreType.DMA((N_DEV - 1,)),
            pltpu.SemaphoreType.DMA((N_DEV - 1,)),
            pltpu.SemaphoreType.DMA((N_DEV - 1,)),
            pltpu.SemaphoreType.DMA((N_DEV - 1,)),
        ],
        compiler_params=pltpu.CompilerParams(collective_id=0),
    )(x, Wq, K_r, V_r, Wo)


# baseline (device time: 164688 ns/iter reference)
import jax
import jax.numpy as jnp
from jax import lax
from jax.experimental import pallas as pl
from jax.experimental.pallas import tpu as pltpu

N_DEV = 4
B = 2
SQ = 512
SKV = 512
HG = 8
DH = 64
D_MODEL = 768
BLK = 64


def kernel(x, Wq, K_ext, V_ext, Wo):
    K_r = jnp.transpose(K_ext.reshape(B, SKV, N_DEV, HG, DH), (2, 0, 3, 1, 4))
    V_r = jnp.transpose(V_ext.reshape(B, SKV, N_DEV, HG, DH), (2, 0, 3, 1, 4))

    def body(x_ref, wq_ref, k_ref, v_ref, wo_ref, out_ref,
             comm_wq, comm_wo, send_wq, recv_wq, send_wo, recv_wo):
        my = lax.axis_index("i")
        left = (my + N_DEV - 1) % N_DEV
        right = (my + 1) % N_DEV

        barrier_sem = pltpu.get_barrier_semaphore()
        for nbr in (left, right):
            pl.semaphore_signal(
                barrier_sem, inc=1,
                device_id=(nbr,), device_id_type=pl.DeviceIdType.MESH,
            )
        pl.semaphore_wait(barrier_sem, 2)

        comm_wq[0] = wq_ref[...]
        comm_wo[0] = wo_ref[...]

        ri = lax.broadcasted_iota(jnp.int32, (SQ, SKV), 0)
        ci = lax.broadcasted_iota(jnp.int32, (SQ, SKV), 1)
        qb = my * (SQ // BLK) + ri // BLK
        kb = ci // BLK
        mask = (qb == kb) | (kb == 0) | ((qb + kb) % 3 == 0)
        bias = jnp.where(mask, 0.0, -1e9).astype(jnp.float32)

        def contribution(slot, origin):
            wq = comm_wq[slot]
            wo = comm_wo[slot]
            for b in range(B):
                q = jnp.dot(x_ref[b], wq, preferred_element_type=jnp.float32)
                acc = None
                for hh in range(HG):
                    qh = q[:, hh * DH:(hh + 1) * DH]
                    kh = k_ref[origin, b, hh]
                    vh = v_ref[origin, b, hh]
                    s = lax.dot_general(
                        qh, kh, (((1,), (1,)), ((), ())),
                        preferred_element_type=jnp.float32,
                    ) * 0.125 + bias
                    m = jnp.max(s, axis=1, keepdims=True)
                    e = jnp.exp(s - m)
                    w = e / jnp.sum(e, axis=1, keepdims=True)
                    ctx = jnp.dot(w, vh, preferred_element_type=jnp.float32)
                    part = jnp.dot(ctx, wo[hh * DH:(hh + 1) * DH, :],
                                   preferred_element_type=jnp.float32)
                    acc = part if acc is None else acc + part
                if slot == 0:
                    out_ref[b] = acc
                else:
                    out_ref[b] = out_ref[b] + acc

        for h in range(N_DEV - 1):
            rq = pltpu.make_async_remote_copy(
                src_ref=comm_wq.at[h], dst_ref=comm_wq.at[h + 1],
                send_sem=send_wq.at[h], recv_sem=recv_wq.at[h],
                device_id=(right,), device_id_type=pl.DeviceIdType.MESH,
            )
            ro = pltpu.make_async_remote_copy(
                src_ref=comm_wo.at[h], dst_ref=comm_wo.at[h + 1],
                send_sem=send_wo.at[h], recv_sem=recv_wo.at[h],
                device_id=(right,), device_id_type=pl.DeviceIdType.MESH,
            )
            rq.start()
            ro.start()
            contribution(h, (my + N_DEV - h) % N_DEV)
            rq.wait()
            ro.wait()

        contribution(N_DEV - 1, (my + 1) % N_DEV)

    return pl.pallas_call(
        body,
        out_shape=jax.ShapeDtypeStruct((B, SQ, D_MODEL), jnp.float32),
        in_specs=[pl.BlockSpec(memory_space=pltpu.VMEM)] * 5,
        out_specs=pl.BlockSpec(memory_space=pltpu.VMEM),
        scratch_shapes=[
            pltpu.VMEM((N_DEV, D_MODEL, HG * DH), jnp.float32),
            pltpu.VMEM((N_DEV, HG * DH, D_MODEL), jnp.float32),
            pltpu.SemaphoreType.DMA((N_DEV - 1,)),
            pltpu.SemaphoreType.DMA((N_DEV - 1,)),
            pltpu.SemaphoreType.DMA((N_DEV - 1,)),
            pltpu.SemaphoreType.DMA((N_DEV - 1,)),
        ],
        compiler_params=pltpu.CompilerParams(
            collective_id=0, vmem_limit_bytes=100 * 1024 * 1024,
        ),
    )(x, Wq, K_r, V_r, Wo)


# device time: 101404 ns/iter; 1.6241x vs baseline; 1.6241x over previous
import jax
import jax.numpy as jnp
from jax import lax
from jax.experimental import pallas as pl
from jax.experimental.pallas import tpu as pltpu

N_DEV = 4
B = 2
SQ = 512
SKV = 512
HG = 8
DH = 64
D_MODEL = 768
BLK = 64


def kernel(x, Wq, K_ext, V_ext, Wo):
    K_r = jnp.transpose(
        K_ext.reshape(B, SKV, N_DEV, HG, DH), (2, 0, 3, 1, 4)
    ).astype(jnp.bfloat16)
    V_r = jnp.transpose(
        V_ext.reshape(B, SKV, N_DEV, HG, DH), (2, 0, 3, 1, 4)
    ).astype(jnp.bfloat16)
    x16 = x.astype(jnp.bfloat16)
    Wq16 = Wq.astype(jnp.bfloat16)
    Wo16 = Wo.astype(jnp.bfloat16)

    def body(x_ref, wq_ref, k_ref, v_ref, wo_ref, out_ref,
             comm_wq, comm_wo, send_wq, recv_wq, send_wo, recv_wo):
        my = lax.axis_index("i")
        left = (my + N_DEV - 1) % N_DEV
        right = (my + 1) % N_DEV

        barrier_sem = pltpu.get_barrier_semaphore()
        for nbr in (left, right):
            pl.semaphore_signal(
                barrier_sem, inc=1,
                device_id=(nbr,), device_id_type=pl.DeviceIdType.MESH,
            )
        pl.semaphore_wait(barrier_sem, 2)

        comm_wq[0] = wq_ref[...]
        comm_wo[0] = wo_ref[...]

        ri = lax.broadcasted_iota(jnp.int32, (SQ, SKV), 0)
        ci = lax.broadcasted_iota(jnp.int32, (SQ, SKV), 1)
        qb = my * (SQ // BLK) + ri // BLK
        kb = ci // BLK
        mask = (qb == kb) | (kb == 0) | ((qb + kb) % 3 == 0)
        bias = jnp.where(mask, 0.0, -1e9).astype(jnp.float32)

        x2 = x_ref[...].reshape(B * SQ, D_MODEL)

        def contribution(slot, origin):
            wq = comm_wq[slot]
            wo = comm_wo[slot]
            q_all = jnp.dot(x2, wq, preferred_element_type=jnp.float32)
            q_all = (q_all * 0.125).astype(jnp.bfloat16)
            for b in range(B):
                q = q_all[b * SQ:(b + 1) * SQ]
                ctx = []
                for hh in range(HG):
                    qh = q[:, hh * DH:(hh + 1) * DH]
                    kh = k_ref[origin, b, hh]
                    vh = v_ref[origin, b, hh]
                    s = lax.dot_general(
                        qh, kh, (((1,), (1,)), ((), ())),
                        preferred_element_type=jnp.float32,
                    ) + bias
                    m = jnp.max(s, axis=1, keepdims=True)
                    e = jnp.exp(s - m)
                    w = (e / jnp.sum(e, axis=1, keepdims=True)).astype(
                        jnp.bfloat16)
                    ctx.append(jnp.dot(w, vh,
                                       preferred_element_type=jnp.float32))
                ctx = jnp.concatenate(ctx, axis=1).astype(jnp.bfloat16)
                part = jnp.dot(ctx, wo, preferred_element_type=jnp.float32)
                if slot == 0:
                    out_ref[b] = part
                else:
                    out_ref[b] = out_ref[b] + part

        for h in range(N_DEV - 1):
            rq = pltpu.make_async_remote_copy(
                src_ref=comm_wq.at[h], dst_ref=comm_wq.at[h + 1],
                send_sem=send_wq.at[h], recv_sem=recv_wq.at[h],
                device_id=(right,), device_id_type=pl.DeviceIdType.MESH,
            )
            ro = pltpu.make_async_remote_copy(
                src_ref=comm_wo.at[h], dst_ref=comm_wo.at[h + 1],
                send_sem=send_wo.at[h], recv_sem=recv_wo.at[h],
                device_id=(right,), device_id_type=pl.DeviceIdType.MESH,
            )
            rq.start()
            ro.start()
            contribution(h, (my + N_DEV - h) % N_DEV)
            rq.wait()
            ro.wait()

        contribution(N_DEV - 1, (my + 1) % N_DEV)

    return pl.pallas_call(
        body,
        out_shape=jax.ShapeDtypeStruct((B, SQ, D_MODEL), jnp.float32),
        in_specs=[pl.BlockSpec(memory_space=pltpu.VMEM)] * 5,
        out_specs=pl.BlockSpec(memory_space=pltpu.VMEM),
        scratch_shapes=[
            pltpu.VMEM((N_DEV, D_MODEL, HG * DH), jnp.bfloat16),
            pltpu.VMEM((N_DEV, HG * DH, D_MODEL), jnp.bfloat16),
            pltpu.SemaphoreType.DMA((N_DEV - 1,)),
            pltpu.SemaphoreType.DMA((N_DEV - 1,)),
            pltpu.SemaphoreType.DMA((N_DEV - 1,)),
            pltpu.SemaphoreType.DMA((N_DEV - 1,)),
        ],
        compiler_params=pltpu.CompilerParams(
            collective_id=0, vmem_limit_bytes=100 * 1024 * 1024,
        ),
    )(x16, Wq16, K_r, V_r, Wo16)


# device time: 98999 ns/iter; 1.6635x vs baseline; 1.0243x over previous
import jax
import jax.numpy as jnp
from jax import lax
from jax.experimental import pallas as pl
from jax.experimental.pallas import tpu as pltpu

N_DEV = 4
B = 2
SQ = 512
SKV = 512
HG = 8
DH = 64
D_MODEL = 768
BLK = 64


def kernel(x, Wq, K_ext, V_ext, Wo):
    K_r = jnp.transpose(
        K_ext.reshape(B, SKV, N_DEV, HG, DH), (2, 0, 3, 1, 4)
    ).astype(jnp.bfloat16)
    V_r = jnp.transpose(
        V_ext.reshape(B, SKV, N_DEV, HG, DH), (2, 0, 3, 1, 4)
    ).astype(jnp.bfloat16)
    x16 = x.astype(jnp.bfloat16)
    Wq16 = Wq.astype(jnp.bfloat16)
    Wo16 = Wo.astype(jnp.bfloat16)

    def body(x_ref, wq_ref, k_ref, v_ref, wo_ref, out_ref,
             comm_wq, comm_wo, send_wq, recv_wq, send_wo, recv_wo):
        my = lax.axis_index("i")
        left = (my + N_DEV - 1) % N_DEV
        right = (my + 1) % N_DEV

        barrier_sem = pltpu.get_barrier_semaphore()
        for nbr in (left, right):
            pl.semaphore_signal(
                barrier_sem, inc=1,
                device_id=(nbr,), device_id_type=pl.DeviceIdType.MESH,
            )
        pl.semaphore_wait(barrier_sem, 2)

        comm_wq[0] = wq_ref[...]
        comm_wo[0] = wo_ref[...]

        ri = lax.broadcasted_iota(jnp.int32, (SQ, SKV), 0)
        ci = lax.broadcasted_iota(jnp.int32, (SQ, SKV), 1)
        qb = my * (SQ // BLK) + ri // BLK
        kb = ci // BLK
        mask = (qb == kb) | (kb == 0) | ((qb + kb) % 3 == 0)
        bias = jnp.where(mask, 0.0, -30.0).astype(jnp.float32)

        x2 = x_ref[...].reshape(B * SQ, D_MODEL)

        def contribution(slot, origin):
            wq = comm_wq[slot]
            wo = comm_wo[slot]
            q_all = jnp.dot(x2, wq, preferred_element_type=jnp.float32)
            q_all = (q_all * 0.125).astype(jnp.bfloat16)
            for b in range(B):
                q = q_all[b * SQ:(b + 1) * SQ]
                ctx = []
                for hh in range(HG):
                    qh = q[:, hh * DH:(hh + 1) * DH]
                    kh = k_ref[origin, b, hh]
                    vh = v_ref[origin, b, hh]
                    s = lax.dot_general(
                        qh, kh, (((1,), (1,)), ((), ())),
                        preferred_element_type=jnp.float32,
                    ) + bias
                    e = jnp.exp(s)
                    inv = 1.0 / jnp.sum(e, axis=1, keepdims=True)
                    c = jnp.dot(e.astype(jnp.bfloat16), vh,
                                preferred_element_type=jnp.float32)
                    ctx.append(c * inv)
                ctx = jnp.concatenate(ctx, axis=1).astype(jnp.bfloat16)
                part = jnp.dot(ctx, wo, preferred_element_type=jnp.float32)
                if slot == 0:
                    out_ref[b] = part
                else:
                    out_ref[b] = out_ref[b] + part

        for h in range(N_DEV - 1):
            rq = pltpu.make_async_remote_copy(
                src_ref=comm_wq.at[h], dst_ref=comm_wq.at[h + 1],
                send_sem=send_wq.at[h], recv_sem=recv_wq.at[h],
                device_id=(right,), device_id_type=pl.DeviceIdType.MESH,
            )
            ro = pltpu.make_async_remote_copy(
                src_ref=comm_wo.at[h], dst_ref=comm_wo.at[h + 1],
                send_sem=send_wo.at[h], recv_sem=recv_wo.at[h],
                device_id=(right,), device_id_type=pl.DeviceIdType.MESH,
            )
            rq.start()
            ro.start()
            contribution(h, (my + N_DEV - h) % N_DEV)
            rq.wait()
            ro.wait()

        contribution(N_DEV - 1, (my + 1) % N_DEV)

    return pl.pallas_call(
        body,
        out_shape=jax.ShapeDtypeStruct((B, SQ, D_MODEL), jnp.float32),
        in_specs=[pl.BlockSpec(memory_space=pltpu.VMEM)] * 5,
        out_specs=pl.BlockSpec(memory_space=pltpu.VMEM),
        scratch_shapes=[
            pltpu.VMEM((N_DEV, D_MODEL, HG * DH), jnp.bfloat16),
            pltpu.VMEM((N_DEV, HG * DH, D_MODEL), jnp.bfloat16),
            pltpu.SemaphoreType.DMA((N_DEV - 1,)),
            pltpu.SemaphoreType.DMA((N_DEV - 1,)),
            pltpu.SemaphoreType.DMA((N_DEV - 1,)),
            pltpu.SemaphoreType.DMA((N_DEV - 1,)),
        ],
        compiler_params=pltpu.CompilerParams(
            collective_id=0, vmem_limit_bytes=100 * 1024 * 1024,
        ),
    )(x16, Wq16, K_r, V_r, Wo16)


# device time: 94290 ns/iter; 1.7466x vs baseline; 1.0499x over previous
import jax
import jax.numpy as jnp
from jax import lax
from jax.experimental import pallas as pl
from jax.experimental.pallas import tpu as pltpu

N_DEV = 4
B = 2
SQ = 512
SKV = 512
HG = 8
DH = 64
D_MODEL = 768
BLK = 64


def kernel(x, Wq, K_ext, V_ext, Wo):
    K_r = jnp.transpose(
        K_ext.reshape(B, SKV, N_DEV, HG, DH), (2, 0, 3, 1, 4)
    ).astype(jnp.bfloat16)
    V_r = jnp.transpose(
        V_ext.reshape(B, SKV, N_DEV, HG, DH), (2, 0, 3, 1, 4)
    ).astype(jnp.bfloat16)
    V_aug = jnp.concatenate(
        [V_r, jnp.ones((N_DEV, B, HG, SKV, 1), jnp.bfloat16)], axis=-1
    )
    x16 = (x * 0.125).astype(jnp.bfloat16)
    Wq16 = Wq.astype(jnp.bfloat16)
    Wo16 = Wo.astype(jnp.bfloat16)

    def body(x_ref, wq_ref, k_ref, v_ref, wo_ref, out_ref,
             comm_wq, comm_wo, send_wq, recv_wq, send_wo, recv_wo):
        my = lax.axis_index("i")
        left = (my + N_DEV - 1) % N_DEV
        right = (my + 1) % N_DEV

        barrier_sem = pltpu.get_barrier_semaphore()
        for nbr in (left, right):
            pl.semaphore_signal(
                barrier_sem, inc=1,
                device_id=(nbr,), device_id_type=pl.DeviceIdType.MESH,
            )
        pl.semaphore_wait(barrier_sem, 2)

        comm_wq[0] = wq_ref[...]
        comm_wo[0] = wo_ref[...]

        ri = lax.broadcasted_iota(jnp.int32, (SQ, SKV), 0)
        ci = lax.broadcasted_iota(jnp.int32, (SQ, SKV), 1)
        qb = my * (SQ // BLK) + ri // BLK
        kb = ci // BLK
        mask = (qb == kb) | (kb == 0) | ((qb + kb) % 3 == 0)
        bias = jnp.where(mask, 0.0, -30.0).astype(jnp.float32)

        x2 = x_ref[...].reshape(B * SQ, D_MODEL)

        def contribution(slot, origin):
            wq = comm_wq[slot]
            wo = comm_wo[slot]
            q_all = jnp.dot(
                x2, wq, preferred_element_type=jnp.float32
            ).astype(jnp.bfloat16)
            for b in range(B):
                q = q_all[b * SQ:(b + 1) * SQ]
                ctx = []
                for hh in range(HG):
                    qh = q[:, hh * DH:(hh + 1) * DH]
                    kh = k_ref[origin, b, hh]
                    va = v_ref[origin, b, hh]
                    s = lax.dot_general(
                        qh, kh, (((1,), (1,)), ((), ())),
                        preferred_element_type=jnp.float32,
                    )
                    e = jnp.exp(s + bias).astype(jnp.bfloat16)
                    cs = jnp.dot(e, va, preferred_element_type=jnp.float32)
                    inv = 1.0 / cs[:, DH:DH + 1]
                    ctx.append((cs[:, :DH] * inv).astype(jnp.bfloat16))
                ctx = jnp.concatenate(ctx, axis=1)
                part = jnp.dot(ctx, wo, preferred_element_type=jnp.float32)
                if slot == 0:
                    out_ref[b] = part
                else:
                    out_ref[b] = out_ref[b] + part

        for h in range(N_DEV - 1):
            rq = pltpu.make_async_remote_copy(
                src_ref=comm_wq.at[h], dst_ref=comm_wq.at[h + 1],
                send_sem=send_wq.at[h], recv_sem=recv_wq.at[h],
                device_id=(right,), device_id_type=pl.DeviceIdType.MESH,
            )
            ro = pltpu.make_async_remote_copy(
                src_ref=comm_wo.at[h], dst_ref=comm_wo.at[h + 1],
                send_sem=send_wo.at[h], recv_sem=recv_wo.at[h],
                device_id=(right,), device_id_type=pl.DeviceIdType.MESH,
            )
            rq.start()
            ro.start()
            contribution(h, (my + N_DEV - h) % N_DEV)
            rq.wait()
            ro.wait()

        contribution(N_DEV - 1, (my + 1) % N_DEV)

    return pl.pallas_call(
        body,
        out_shape=jax.ShapeDtypeStruct((B, SQ, D_MODEL), jnp.float32),
        in_specs=[pl.BlockSpec(memory_space=pltpu.VMEM)] * 5,
        out_specs=pl.BlockSpec(memory_space=pltpu.VMEM),
        scratch_shapes=[
            pltpu.VMEM((N_DEV, D_MODEL, HG * DH), jnp.bfloat16),
            pltpu.VMEM((N_DEV, HG * DH, D_MODEL), jnp.bfloat16),
            pltpu.SemaphoreType.DMA((N_DEV - 1,)),
            pltpu.SemaphoreType.DMA((N_DEV - 1,)),
            pltpu.SemaphoreType.DMA((N_DEV - 1,)),
            pltpu.SemaphoreType.DMA((N_DEV - 1,)),
        ],
        compiler_params=pltpu.CompilerParams(
            collective_id=0, vmem_limit_bytes=100 * 1024 * 1024,
        ),
    )(x16, Wq16, K_r, V_aug, Wo16)
